# baseline (speedup 1.0000x reference)
import functools

import jax
import jax.numpy as jnp
from jax import lax
from jax.experimental import pallas as pl
from jax.experimental.pallas import tpu as pltpu
from jax.experimental.pallas import tpu_sc as plsc

NC = 2
NS = 16
L = 16
NW = NC * NS

ROWS = 4 * 8192
COLS = 1024
ROWS_W = ROWS // NW
RBLK = 8
NBLK = ROWS_W // RBLK
UNROLL = 8
NIN = 4
NOUT = 4

_mesh = plsc.VectorSubcoreMesh(core_axis_name="c", subcore_axis_name="s")


@functools.partial(
    pl.kernel,
    out_type=jax.ShapeDtypeStruct((ROWS, COLS), jnp.float32),
    mesh=_mesh,
    compiler_params=pltpu.CompilerParams(needs_layout_passes=False),
    scratch_types=[
        pltpu.VMEM((256,), jnp.float32),
        [pltpu.VMEM((RBLK, COLS), jnp.int32) for _ in range(NIN)],
        [pltpu.VMEM((RBLK, COLS), jnp.float32) for _ in range(NOUT)],
        pltpu.VMEM((L,), jnp.float32),
        pltpu.VMEM((L,), jnp.float32),
        [pltpu.SemaphoreType.DMA for _ in range(NIN)],
        [pltpu.SemaphoreType.DMA for _ in range(NOUT)],
    ],
)
def _sc_lut_kernel(data_hbm, scale_hbm, dscale_hbm, out_hbm,
                   table_v, idx_bufs, out_bufs, sv_v, dv_v,
                   in_sems, out_sems):
    wid = lax.axis_index("s") * NC + lax.axis_index("c")
    base = wid * ROWS_W

    pltpu.sync_copy(scale_hbm, sv_v)
    pltpu.sync_copy(dscale_hbm, dv_v)
    sv = sv_v[...]
    dv = dv_v[...]

    lane = lax.iota(jnp.int32, L)
    for i in range(256 // L):
        qx = (lane + (i * L - 128)).astype(jnp.float32)
        x = qx * dv
        y = 1.0 / (1.0 + jnp.exp(-x))
        t = y / sv
        q = jnp.minimum((t + 0.5).astype(jnp.int32), 127)
        table_v[pl.ds(i * L, L)] = q.astype(jnp.float32) * sv

    def start_in(slot, b):
        pltpu.async_copy(data_hbm.at[pl.ds(base + b * RBLK, RBLK), :],
                         idx_bufs[slot], in_sems[slot])

    for k in range(NIN):
        start_in(k, k)

    def quad_body(i, _):
        for k in range(NIN):
            b = NIN * i + k
            ko = k % NOUT
            ib, ob = idx_bufs[k], out_bufs[ko]
            pltpu.make_async_copy(data_hbm.at[pl.ds(0, RBLK), :], ib,
                                  in_sems[k]).wait()

            @pl.when(b >= NOUT)
            def _wait_out(ob=ob, ko=ko):
                pltpu.make_async_copy(ob, out_hbm.at[pl.ds(0, RBLK), :],
                                      out_sems[ko]).wait()

            def row_body(r, _, ib=ib, ob=ob):
                for g in range(COLS // (L * UNROLL)):
                    c0 = g * L * UNROLL
                    ids = [ib[r, pl.ds(c0 + u * L, L)]
                           for u in range(UNROLL)]
                    if g == 0:
                        gs = []
                        for v in ids:
                            qx = (v - 128).astype(jnp.float32)
                            x = qx * dv
                            y = 1.0 / (1.0 + jnp.exp(-x))
                            t = y / sv
                            q = jnp.minimum((t + 0.5).astype(jnp.int32),
                                            127)
                            gs.append(q.astype(jnp.float32) * sv)
                    else:
                        gs = [plsc.load_gather(table_v, [v]) for v in ids]
                    for u in range(UNROLL):
                        ob[r, pl.ds(c0 + u * L, L)] = gs[u]
                return 0

            lax.fori_loop(0, RBLK, row_body, 0)

            pltpu.async_copy(ob, out_hbm.at[pl.ds(base + b * RBLK, RBLK), :],
                             out_sems[ko])

            @pl.when(b + NIN < NBLK)
            def _prefetch(k=k, b=b):
                start_in(k, b + NIN)
        return 0

    lax.fori_loop(0, NBLK // NIN, quad_body, 0)

    for ko in range(NOUT):
        pltpu.make_async_copy(out_bufs[ko], out_hbm.at[pl.ds(0, RBLK), :],
                              out_sems[ko]).wait()


def kernel(data, scale, data_scale):
    data2 = data.reshape(ROWS, COLS)
    s16 = jnp.broadcast_to(scale.astype(jnp.float32), (L,))
    d16 = jnp.broadcast_to(data_scale.astype(jnp.float32), (L,))
    out = _sc_lut_kernel(data2, s16, d16)
    return out.reshape(data.shape)
